# baseline (speedup 1.0000x reference)
import dataclasses
import functools

import jax
import jax.numpy as jnp
from jax import lax
from jax.experimental import pallas as pl
from jax.experimental.pallas import tpu as pltpu
from jax.experimental.pallas import tpu_sc as plsc

N = 10000
E = 160000
F = 256
HID = 256
HEADS = 2
OUT = 128

_NB = 2000
_EB = 2000


def _tc1_body(x_ref, wu_ref, wv_ref, b1_ref, u_ref, v_ref):
    x = x_ref[...]
    u_ref[...] = jnp.dot(x, wu_ref[...], preferred_element_type=jnp.float32) + b1_ref[...]
    v_ref[...] = jnp.dot(x, wv_ref[...], preferred_element_type=jnp.float32)


def _tc1(x, wu, wv, b1):
    return pl.pallas_call(
        _tc1_body,
        grid=(N // _NB,),
        in_specs=[
            pl.BlockSpec((_NB, F), lambda i: (i, 0)),
            pl.BlockSpec((F, HID), lambda i: (0, 0)),
            pl.BlockSpec((F, HID), lambda i: (0, 0)),
            pl.BlockSpec((1, HID), lambda i: (0, 0)),
        ],
        out_specs=[
            pl.BlockSpec((_NB, HID), lambda i: (i, 0)),
            pl.BlockSpec((_NB, HID), lambda i: (i, 0)),
        ],
        out_shape=[
            jax.ShapeDtypeStruct((N, HID), jnp.float32),
            jax.ShapeDtypeStruct((N, HID), jnp.float32),
        ],
    )(x, wu, wv, b1.reshape(1, HID))


def _tc2_body(g1_ref, g2_ref, w2_ref, b2_ref, m_ref):
    g = jnp.maximum(g1_ref[...] + g2_ref[...], 0.0)
    m_ref[...] = jnp.dot(g, w2_ref[...], preferred_element_type=jnp.float32) + b2_ref[...]


def _tc2(g1, g2, w2, b2):
    return pl.pallas_call(
        _tc2_body,
        grid=(E // _EB,),
        in_specs=[
            pl.BlockSpec((_EB, HID), lambda i: (i, 0)),
            pl.BlockSpec((_EB, HID), lambda i: (i, 0)),
            pl.BlockSpec((HID, HID), lambda i: (0, 0)),
            pl.BlockSpec((1, HID), lambda i: (0, 0)),
        ],
        out_specs=pl.BlockSpec((_EB, HID), lambda i: (i, 0)),
        out_shape=jax.ShapeDtypeStruct((E, HID), jnp.float32),
    )(g1, g2, w2, b2.reshape(1, HID))


_MESH = plsc.VectorSubcoreMesh(core_axis_name="c", subcore_axis_name="s")
_CP = pltpu.CompilerParams()
if "needs_layout_passes" in pltpu.CompilerParams.__dataclass_fields__:
    _CP = dataclasses.replace(_CP, needs_layout_passes=False)
_NW = 32
_EPW = E // _NW
_GB1 = 40


def _sc_gather2(u, v, src, dst):
    @functools.partial(
        pl.kernel,
        out_type=[jax.ShapeDtypeStruct((E, HID), jnp.float32),
                  jax.ShapeDtypeStruct((E, HID), jnp.float32)],
        mesh=_MESH,
        scratch_types=[
            pltpu.VMEM((_GB1,), jnp.int32),
            pltpu.VMEM((_GB1,), jnp.int32),
            pltpu.VMEM((_GB1, HID), jnp.float32),
            pltpu.VMEM((_GB1, HID), jnp.float32),
            pltpu.VMEM((_GB1,), jnp.int32),
            pltpu.VMEM((_GB1,), jnp.int32),
            pltpu.VMEM((_GB1, HID), jnp.float32),
            pltpu.VMEM((_GB1, HID), jnp.float32),
            pltpu.SemaphoreType.DMA,
            pltpu.SemaphoreType.DMA,
            pltpu.SemaphoreType.DMA,
            pltpu.SemaphoreType.DMA,
            pltpu.SemaphoreType.DMA,
            pltpu.SemaphoreType.DMA,
        ],
    )
    def k(u_hbm, v_hbm, src_hbm, dst_hbm, g1_hbm, g2_hbm,
          idxdA, idxsA, urA, vrA, idxdB, idxsB, urB, vrB,
          semIA, semGA, semWA, semIB, semGB, semWB):
        wid = lax.axis_index("s") * 2 + lax.axis_index("c")
        base = wid * _EPW
        nb = _EPW // _GB1

        sets = {
            0: (idxdA, idxsA, urA, vrA, semIA, semGA, semWA),
            1: (idxdB, idxsB, urB, vrB, semIB, semGB, semWB),
        }

        def stage_idx(b, s):
            idxd, idxs, _, _, semI, _, _ = sets[s]
            off = base + b * _GB1
            pltpu.async_copy(dst_hbm.at[pl.ds(off, _GB1)], idxd, semI)
            pltpu.async_copy(src_hbm.at[pl.ds(off, _GB1)], idxs, semI)

        def idx_wait(s):
            idxd, idxs, _, _, semI, _, _ = sets[s]
            pltpu.make_async_copy(dst_hbm.at[pl.ds(0, _GB1)], idxd, semI).wait()
            pltpu.make_async_copy(src_hbm.at[pl.ds(0, _GB1)], idxs, semI).wait()

        def stage_gather(s):
            idxd, idxs, ur, vr, _, semG, _ = sets[s]
            pltpu.async_copy(u_hbm.at[idxd], ur, semG)
            pltpu.async_copy(v_hbm.at[idxs], vr, semG)

        def gather_wait(s):
            idxd, idxs, ur, vr, _, semG, _ = sets[s]
            pltpu.make_async_copy(u_hbm.at[idxd], ur, semG).wait()
            pltpu.make_async_copy(v_hbm.at[idxs], vr, semG).wait()

        def stage_wb(b, s):
            _, _, ur, vr, _, _, semW = sets[s]
            off = base + b * _GB1
            pltpu.async_copy(ur, g1_hbm.at[pl.ds(off, _GB1)], semW)
            pltpu.async_copy(vr, g2_hbm.at[pl.ds(off, _GB1)], semW)

        def wb_wait(s):
            _, _, ur, vr, _, _, semW = sets[s]
            pltpu.make_async_copy(ur, g1_hbm.at[pl.ds(0, _GB1)], semW).wait()
            pltpu.make_async_copy(vr, g2_hbm.at[pl.ds(0, _GB1)], semW).wait()

        stage_idx(0, 0)

        def pair(p, carry):
            b = p * 2

            @pl.when(p > 0)
            def _():
                wb_wait(0)

            idx_wait(0)
            stage_gather(0)
            stage_idx(b + 1, 1)

            @pl.when(p > 0)
            def _():
                wb_wait(1)

            gather_wait(0)
            stage_wb(b, 0)
            idx_wait(1)
            stage_gather(1)
            stage_idx(b + 2, 0)
            gather_wait(1)
            stage_wb(b + 1, 1)
            return carry

        lax.fori_loop(0, nb // 2, pair, jnp.int32(0))
        wb_wait(0)
        idx_wait(0)
        stage_gather(0)
        gather_wait(0)
        stage_wb(nb - 1, 0)
        wb_wait(1)
        wb_wait(0)

    return k(u, v, src, dst)


_NOWN = 320
_NPAD = _NW * _NOWN
_CH = 2000
_GB = 16
_QMAX = 3200
_QTH = _QMAX - _CH - _GB


def _iota16():
    return lax.iota(jnp.int32, 16)


def _lane_splat(v16, k):
    dn = lax.GatherDimensionNumbers(
        offset_dims=(), collapsed_slice_dims=(0,), start_index_map=(0,))
    return lax.gather(v16, jnp.full((16, 1), k, jnp.int32), dn, (1,),
                      mode=lax.GatherScatterMode.PROMISE_IN_BOUNDS)


def _sc_segmax(m, dst):
    @functools.partial(
        pl.kernel,
        out_type=jax.ShapeDtypeStruct((_NPAD, HID), jnp.float32),
        mesh=_MESH,
        compiler_params=_CP,
        scratch_types=[
            pltpu.VMEM((_NOWN + 1, HID), jnp.float32),
            pltpu.VMEM((_CH,), jnp.int32),
            pltpu.VMEM((_CH,), jnp.int32),
            pltpu.VMEM((_QMAX,), jnp.int32),
            pltpu.VMEM((_QMAX,), jnp.int32),
            pltpu.VMEM((_GB, HID), jnp.float32),
            pltpu.VMEM((_GB, HID), jnp.float32),
            pltpu.VMEM((_GB, HID), jnp.float32),
            pltpu.SemaphoreType.DMA,
            pltpu.SemaphoreType.DMA,
            pltpu.SemaphoreType.DMA,
            pltpu.SemaphoreType.DMA,
            pltpu.SemaphoreType.DMA,
        ],
    )
    def k(m_hbm, dst_hbm, out_hbm, acc, dvmA, dvmB, qeid, qlid,
          rows0, rows1, rows2, semA, semB, semG0, semG1, semG2):
        wid = lax.axis_index("s") * 2 + lax.axis_index("c")
        lo = wid * _NOWN
        iot = _iota16()
        ninf = jnp.full((16,), -jnp.inf, jnp.float32)

        @pl.loop(0, _NOWN + 1)
        def _(r):
            rv = jnp.full((16,), 1, jnp.int32) * r

            @pl.loop(0, HID // 16)
            def _(cc):
                plsc.store_scatter(acc, [rv, cc * 16 + iot], ninf)

        def gstart(b, rows, sem):
            pltpu.async_copy(m_hbm.at[qeid.at[pl.ds(b * _GB, _GB)]], rows, sem)

        def gwait(rows, sem):
            pltpu.make_async_copy(m_hbm.at[qeid.at[pl.ds(0, _GB)]], rows, sem).wait()

        def rmw(boff, rows):
            @pl.loop(0, _GB // 4)
            def _(rr):
                r = rr * 4
                one = jnp.full((16,), 1, jnp.int32)
                rsp = [plsc.load_gather(qlid, [one * (boff + r + j)])
                       for j in range(4)]
                ksp = [one * (r + j) for j in range(4)]
                for cc in range(HID // 16):
                    colv = cc * 16 + iot
                    mrow = [plsc.load_gather(rows, [ksp[j], colv])
                            for j in range(4)]
                    for j in range(4):
                        cur = plsc.load_gather(acc, [rsp[j], colv])
                        plsc.store_scatter(acc, [rsp[j], colv],
                                           jnp.maximum(cur, mrow[j]))

        rb = ((rows0, semG0), (rows1, semG1), (rows2, semG2))

        def drain(qn):
            plsc.store_scatter(qeid, [qn + iot], iot * 64)
            plsc.store_scatter(qlid, [qn + iot], jnp.full((16,), _NOWN, jnp.int32))
            nb = (qn + _GB - 1) // _GB
            for j, (rows, sem) in enumerate(rb):
                @pl.when(nb > j)
                def _(rows=rows, sem=sem, j=j):
                    gstart(j, rows, sem)

            def body(p, carry):
                for j, (rows, sem) in enumerate(rb):
                    b = p * 3 + j

                    @pl.when(b < nb)
                    def _(rows=rows, sem=sem, b=b):
                        gwait(rows, sem)
                        rmw(b * _GB, rows)

                        @pl.when(b + 3 < nb)
                        def _():
                            gstart(b + 3, rows, sem)
                return carry

            lax.fori_loop(0, (nb + 2) // 3, body, jnp.int32(0))

        def scanchunk(c, dvm, qn0):
            def scan(i, qn):
                d16 = dvm[pl.ds(i * 16, 16)]
                msk = (d16 >= lo) & (d16 < lo + _NOWN)
                plsc.store_compressed(qeid.at[pl.ds(qn, 16)],
                                      c * _CH + i * 16 + iot, mask=msk)
                plsc.store_compressed(qlid.at[pl.ds(qn, 16)], d16 - lo, mask=msk)
                return qn + plsc.all_reduce_population_count(msk)[0]

            return lax.fori_loop(0, _CH // 16, scan, qn0)

        def maybe_drain(qn):
            @pl.when(qn > _QTH)
            def _():
                drain(qn)

            return jnp.where(qn > _QTH, jnp.int32(0), qn)

        def start(c, dvm, sem):
            pltpu.async_copy(dst_hbm.at[pl.ds(c * _CH, _CH)], dvm, sem)

        def wait(dvm, sem):
            pltpu.make_async_copy(dst_hbm.at[pl.ds(0, _CH)], dvm, sem).wait()

        start(0, dvmA, semA)
        npair = E // _CH // 2

        def pair(p, qn):
            c = p * 2
            start(c + 1, dvmB, semB)
            wait(dvmA, semA)
            qn = scanchunk(c, dvmA, qn)
            qn = maybe_drain(qn)

            @pl.when(p < npair - 1)
            def _():
                start(c + 2, dvmA, semA)

            wait(dvmB, semB)
            qn = scanchunk(c + 1, dvmB, qn)
            return maybe_drain(qn)

        qn = lax.fori_loop(0, npair, pair, jnp.int32(0))

        @pl.when(qn > 0)
        def _():
            drain(qn)

        pltpu.sync_copy(acc.at[pl.ds(0, _NOWN)], out_hbm.at[pl.ds(lo, _NOWN)])

    return k(m, dst)


def _sc_msg(hg_h, als_h, ald_h, src, dst):
    @functools.partial(
        pl.kernel,
        out_type=jax.ShapeDtypeStruct((_NPAD, HID), jnp.float32),
        mesh=_MESH,
        compiler_params=_CP,
        scratch_types=[
            pltpu.VMEM((_NOWN + 1, HID), jnp.float32),
            pltpu.VMEM((N,), jnp.float32),
            pltpu.VMEM((_NOWN,), jnp.float32),
            pltpu.VMEM((16 * _NOWN,), jnp.float32),
            pltpu.VMEM((_NOWN,), jnp.float32),
            pltpu.VMEM((_CH,), jnp.int32),
            pltpu.VMEM((_CH,), jnp.int32),
            pltpu.VMEM((_CH,), jnp.int32),
            pltpu.VMEM((_CH,), jnp.int32),
            pltpu.VMEM((_QMAX,), jnp.int32),
            pltpu.VMEM((_QMAX,), jnp.int32),
            pltpu.VMEM((_QMAX,), jnp.float32),
            pltpu.VMEM((_GB, HID), jnp.float32),
            pltpu.VMEM((_GB, HID), jnp.float32),
            pltpu.VMEM((_GB, HID), jnp.float32),
            pltpu.SemaphoreType.DMA,
            pltpu.SemaphoreType.DMA,
            pltpu.SemaphoreType.DMA,
            pltpu.SemaphoreType.DMA,
            pltpu.SemaphoreType.DMA,
        ],
    )
    def k(hg_hbm, als_hbm, ald_hbm, src_hbm, dst_hbm, out_hbm,
          acc, alsv, aldo, denl, inv, dvmA, svmA, dvmB, svmB,
          qsrc, qlid, qa, rows0, rows1, rows2,
          semA, semB, semG0, semG1, semG2):
        wid = lax.axis_index("s") * 2 + lax.axis_index("c")
        lo = wid * _NOWN
        iot = _iota16()
        zero16 = jnp.zeros((16,), jnp.float32)

        pltpu.sync_copy(als_hbm, alsv)
        pltpu.sync_copy(ald_hbm.at[pl.ds(lo, _NOWN)], aldo)

        @pl.loop(0, _NOWN)
        def _(i):
            denl[pl.ds(i * 16, 16)] = zero16

        @pl.loop(0, _NOWN + 1)
        def _(r):
            rv = jnp.full((16,), 1, jnp.int32) * r

            @pl.loop(0, HID // 16)
            def _(cc):
                plsc.store_scatter(acc, [rv, cc * 16 + iot], zero16)

        def start(c, dvm, svm, sem):
            pltpu.async_copy(dst_hbm.at[pl.ds(c * _CH, _CH)], dvm, sem)
            pltpu.async_copy(src_hbm.at[pl.ds(c * _CH, _CH)], svm, sem)

        def wait(dvm, svm, sem):
            pltpu.make_async_copy(dst_hbm.at[pl.ds(0, _CH)], dvm, sem).wait()
            pltpu.make_async_copy(src_hbm.at[pl.ds(0, _CH)], svm, sem).wait()

        def gstart(b, rows, sem):
            pltpu.async_copy(hg_hbm.at[qsrc.at[pl.ds(b * _GB, _GB)]], rows, sem)

        def gwait(rows, sem):
            pltpu.make_async_copy(hg_hbm.at[qsrc.at[pl.ds(0, _GB)]], rows, sem).wait()

        def rmw(boff, rows):
            @pl.loop(0, _GB // 4)
            def _(rr):
                r = rr * 4
                one = jnp.full((16,), 1, jnp.int32)
                rsp = [plsc.load_gather(qlid, [one * (boff + r + j)])
                       for j in range(4)]
                asp = [plsc.load_gather(qa, [one * (boff + r + j)])
                       for j in range(4)]
                ksp = [one * (r + j) for j in range(4)]
                for cc in range(HID // 16):
                    colv = cc * 16 + iot
                    mrow = [plsc.load_gather(rows, [ksp[j], colv])
                            for j in range(4)]
                    for j in range(4):
                        plsc.addupdate_scatter(acc, [rsp[j], colv],
                                               mrow[j] * asp[j])

        rb = ((rows0, semG0), (rows1, semG1), (rows2, semG2))

        def drain(qn):
            plsc.store_scatter(qsrc, [qn + iot], iot * 64)
            plsc.store_scatter(qlid, [qn + iot], jnp.full((16,), _NOWN, jnp.int32))
            plsc.store_scatter(qa, [qn + iot], zero16)
            nb = (qn + _GB - 1) // _GB
            for j, (rows, sem) in enumerate(rb):
                @pl.when(nb > j)
                def _(rows=rows, sem=sem, j=j):
                    gstart(j, rows, sem)

            def body(p, carry):
                for j, (rows, sem) in enumerate(rb):
                    b = p * 3 + j

                    @pl.when(b < nb)
                    def _(rows=rows, sem=sem, b=b):
                        gwait(rows, sem)
                        rmw(b * _GB, rows)

                        @pl.when(b + 3 < nb)
                        def _():
                            gstart(b + 3, rows, sem)
                return carry

            lax.fori_loop(0, (nb + 2) // 3, body, jnp.int32(0))

        def maybe_drain(qn):
            @pl.when(qn > _QTH)
            def _():
                drain(qn)

            return jnp.where(qn > _QTH, jnp.int32(0), qn)

        def scanchunk(c, dvm, svm, qn0):
            def scan(i, qn):
                d16 = dvm[pl.ds(i * 16, 16)]
                s16 = svm[pl.ds(i * 16, 16)]
                msk = (d16 >= lo) & (d16 < lo + _NOWN)
                lidx = jnp.where(msk, d16 - lo, 0)
                e = plsc.load_gather(alsv, [s16]) + plsc.load_gather(aldo, [lidx])
                e = jnp.where(e >= 0.0, e, 0.2 * e)
                ex = jnp.exp(e)
                plsc.addupdate_scatter(denl, [lidx * 16 + iot], ex, mask=msk)
                plsc.store_compressed(qsrc.at[pl.ds(qn, 16)], s16, mask=msk)
                plsc.store_compressed(qlid.at[pl.ds(qn, 16)], lidx, mask=msk)
                plsc.store_compressed(qa.at[pl.ds(qn, 16)], ex, mask=msk)
                return qn + plsc.all_reduce_population_count(msk)[0]

            return lax.fori_loop(0, _CH // 16, scan, qn0)

        start(0, dvmA, svmA, semA)
        npair = E // _CH // 2

        def pair(p, qn):
            c = p * 2
            start(c + 1, dvmB, svmB, semB)
            wait(dvmA, svmA, semA)
            qn = scanchunk(c, dvmA, svmA, qn)
            qn = maybe_drain(qn)

            @pl.when(p < npair - 1)
            def _():
                start(c + 2, dvmA, svmA, semA)

            wait(dvmB, svmB, semB)
            qn = scanchunk(c + 1, dvmB, svmB, qn)
            return maybe_drain(qn)

        qn = lax.fori_loop(0, npair, pair, jnp.int32(0))

        @pl.when(qn > 0)
        def _():
            drain(qn)

        @pl.loop(0, _NOWN // 16)
        def _(i):
            tot = jnp.full((16,), 1e-16, jnp.float32)
            for l in range(16):
                gidx = (i * 16 + iot) * 16 + l
                tot = tot + plsc.load_gather(denl, [gidx])
            inv[pl.ds(i * 16, 16)] = 1.0 / tot

        @pl.loop(0, _NOWN)
        def _(r):
            rv = jnp.full((16,), 1, jnp.int32) * r
            isp = plsc.load_gather(inv, [rv])
            for cc in range(HID // 16):
                colv = cc * 16 + iot
                mrow = plsc.load_gather(acc, [rv, colv])
                plsc.store_scatter(acc, [rv, colv], mrow * isp)

        pltpu.sync_copy(acc.at[pl.ds(0, _NOWN)], out_hbm.at[pl.ds(lo, _NOWN)])

    return k(hg_h, als_h, ald_h, src, dst)


_NB2 = 2048


def _tc3_body(h_ref, wg_ref, ad_ref, hg0_ref, hg1_ref, al_ref):
    h = h_ref[...]
    h = jnp.where(h > -3e38, h, 0.0)
    hg = jnp.dot(h, wg_ref[...], preferred_element_type=jnp.float32)
    hg0_ref[...] = hg[:, :HID]
    hg1_ref[...] = hg[:, HID:]
    al_ref[...] = jnp.dot(hg, ad_ref[...], preferred_element_type=jnp.float32)


def _tc3(h, wg, ad):
    k = h.shape[1]
    return pl.pallas_call(
        _tc3_body,
        grid=(_NPAD // _NB2,),
        in_specs=[
            pl.BlockSpec((_NB2, k), lambda i: (i, 0)),
            pl.BlockSpec((k, HEADS * HID), lambda i: (0, 0)),
            pl.BlockSpec((HEADS * HID, 2 * HEADS), lambda i: (0, 0)),
        ],
        out_specs=[
            pl.BlockSpec((_NB2, HID), lambda i: (i, 0)),
            pl.BlockSpec((_NB2, HID), lambda i: (i, 0)),
            pl.BlockSpec((_NB2, 2 * HEADS), lambda i: (i, 0)),
        ],
        out_shape=[
            jax.ShapeDtypeStruct((_NPAD, HID), jnp.float32),
            jax.ShapeDtypeStruct((_NPAD, HID), jnp.float32),
            jax.ShapeDtypeStruct((_NPAD, 2 * HEADS), jnp.float32),
        ],
    )(h, wg, ad)


def _tc4_body(a0_ref, a1_ref, bg_ref, wg_ref, ad_ref, h1_ref, hg0_ref, hg1_ref, al_ref):
    agg = jnp.concatenate([a0_ref[...], a1_ref[...]], axis=1)
    h1 = jnp.maximum(agg + bg_ref[...], 0.0)
    h1_ref[...] = h1
    hg = jnp.dot(h1, wg_ref[...], preferred_element_type=jnp.float32)
    hg0_ref[...] = hg[:, :HID]
    hg1_ref[...] = hg[:, HID:]
    al_ref[...] = jnp.dot(hg, ad_ref[...], preferred_element_type=jnp.float32)


def _tc4(a0, a1, bg, wg, ad):
    k = HEADS * HID
    return pl.pallas_call(
        _tc4_body,
        grid=(_NPAD // _NB2,),
        in_specs=[
            pl.BlockSpec((_NB2, HID), lambda i: (i, 0)),
            pl.BlockSpec((_NB2, HID), lambda i: (i, 0)),
            pl.BlockSpec((1, k), lambda i: (0, 0)),
            pl.BlockSpec((k, k), lambda i: (0, 0)),
            pl.BlockSpec((k, 2 * HEADS), lambda i: (0, 0)),
        ],
        out_specs=[
            pl.BlockSpec((_NB2, k), lambda i: (i, 0)),
            pl.BlockSpec((_NB2, HID), lambda i: (i, 0)),
            pl.BlockSpec((_NB2, HID), lambda i: (i, 0)),
            pl.BlockSpec((_NB2, 2 * HEADS), lambda i: (i, 0)),
        ],
        out_shape=[
            jax.ShapeDtypeStruct((_NPAD, k), jnp.float32),
            jax.ShapeDtypeStruct((_NPAD, HID), jnp.float32),
            jax.ShapeDtypeStruct((_NPAD, HID), jnp.float32),
            jax.ShapeDtypeStruct((_NPAD, 2 * HEADS), jnp.float32),
        ],
    )(a0, a1, bg.reshape(1, k), wg, ad)


def _tc5_body(a0_ref, a1_ref, bg_ref, h1_ref, wr_ref, br_ref, out_ref):
    agg = jnp.concatenate([a0_ref[...], a1_ref[...]], axis=1)
    h = jnp.maximum(agg + bg_ref[...], 0.0) + h1_ref[...]
    out_ref[...] = jnp.dot(h, wr_ref[...], preferred_element_type=jnp.float32) + br_ref[...]


def _tc5(a0, a1, bg, h1, wr, br):
    k = HEADS * HID
    return pl.pallas_call(
        _tc5_body,
        grid=(_NPAD // _NB2,),
        in_specs=[
            pl.BlockSpec((_NB2, HID), lambda i: (i, 0)),
            pl.BlockSpec((_NB2, HID), lambda i: (i, 0)),
            pl.BlockSpec((1, k), lambda i: (0, 0)),
            pl.BlockSpec((_NB2, k), lambda i: (i, 0)),
            pl.BlockSpec((k, OUT), lambda i: (0, 0)),
            pl.BlockSpec((1, OUT), lambda i: (0, 0)),
        ],
        out_specs=pl.BlockSpec((_NB2, OUT), lambda i: (i, 0)),
        out_shape=jax.ShapeDtypeStruct((_NPAD, OUT), jnp.float32),
    )(a0, a1, bg.reshape(1, k), h1, wr, br.reshape(1, OUT))


def _attn_mats(a_s, a_d):
    k = HEADS * HID
    ad = jnp.zeros((k, 2 * HEADS), jnp.float32)
    for h in range(HEADS):
        ad = ad.at[h * HID:(h + 1) * HID, h].set(a_s[h])
        ad = ad.at[h * HID:(h + 1) * HID, HEADS + h].set(a_d[h])
    return ad


def kernel(x, edge_index, W1, b1, W2, b2, Wg1, as1, ad1, bg1, Wg2, as2, ad2, bg2, Wr, br):
    src = edge_index[0]
    dst = edge_index[1]

    wu = W1[:F] - W1[F:]
    wv = W1[F:]
    ad1m = _attn_mats(as1, ad1)
    ad2m = _attn_mats(as2, ad2)

    u, v = _tc1(x, wu, wv, b1)
    g1, g2 = _sc_gather2(u, v, src, dst)
    m = _tc2(g1, g2, W2, b2)
    h0p = _sc_segmax(m, dst)

    hg1_0, hg1_1, al1 = _tc3(h0p, Wg1, ad1m)
    a1_0 = _sc_msg(hg1_0, al1[:N, 0], al1[:, 2], src, dst)
    a1_1 = _sc_msg(hg1_1, al1[:N, 1], al1[:, 3], src, dst)

    h1, hg2_0, hg2_1, al2 = _tc4(a1_0, a1_1, bg1, Wg2, ad2m)
    a2_0 = _sc_msg(hg2_0, al2[:N, 0], al2[:, 2], src, dst)
    a2_1 = _sc_msg(hg2_1, al2[:N, 1], al2[:, 3], src, dst)

    return _tc5(a2_0, a2_1, bg2, h1, Wr, br)[:N]


# speedup vs baseline: 1.1276x; 1.1276x over previous
import dataclasses
import functools

import jax
import jax.numpy as jnp
from jax import lax
from jax.experimental import pallas as pl
from jax.experimental.pallas import tpu as pltpu
from jax.experimental.pallas import tpu_sc as plsc

N = 10000
E = 160000
F = 256
HID = 256
HEADS = 2
OUT = 128

_NB = 2000
_EB = 2000


def _tc1_body(x_ref, wu_ref, wv_ref, b1_ref, u_ref, v_ref):
    x = x_ref[...]
    u_ref[...] = jnp.dot(x, wu_ref[...], preferred_element_type=jnp.float32) + b1_ref[...]
    v_ref[...] = jnp.dot(x, wv_ref[...], preferred_element_type=jnp.float32)


def _tc1(x, wu, wv, b1):
    return pl.pallas_call(
        _tc1_body,
        grid=(N // _NB,),
        in_specs=[
            pl.BlockSpec((_NB, F), lambda i: (i, 0)),
            pl.BlockSpec((F, HID), lambda i: (0, 0)),
            pl.BlockSpec((F, HID), lambda i: (0, 0)),
            pl.BlockSpec((1, HID), lambda i: (0, 0)),
        ],
        out_specs=[
            pl.BlockSpec((_NB, HID), lambda i: (i, 0)),
            pl.BlockSpec((_NB, HID), lambda i: (i, 0)),
        ],
        out_shape=[
            jax.ShapeDtypeStruct((N, HID), jnp.float32),
            jax.ShapeDtypeStruct((N, HID), jnp.float32),
        ],
    )(x, wu, wv, b1.reshape(1, HID))


def _tc2_body(g1_ref, g2_ref, w2_ref, b2_ref, m_ref):
    g = jnp.maximum(g1_ref[...] + g2_ref[...], 0.0)
    m_ref[...] = jnp.dot(g, w2_ref[...], preferred_element_type=jnp.float32) + b2_ref[...]


def _tc2(g1, g2, w2, b2):
    return pl.pallas_call(
        _tc2_body,
        grid=(E // _EB,),
        in_specs=[
            pl.BlockSpec((_EB, HID), lambda i: (i, 0)),
            pl.BlockSpec((_EB, HID), lambda i: (i, 0)),
            pl.BlockSpec((HID, HID), lambda i: (0, 0)),
            pl.BlockSpec((1, HID), lambda i: (0, 0)),
        ],
        out_specs=pl.BlockSpec((_EB, HID), lambda i: (i, 0)),
        out_shape=jax.ShapeDtypeStruct((E, HID), jnp.float32),
    )(g1, g2, w2, b2.reshape(1, HID))


_MESH = plsc.VectorSubcoreMesh(core_axis_name="c", subcore_axis_name="s")
_CP = pltpu.CompilerParams()
if "needs_layout_passes" in pltpu.CompilerParams.__dataclass_fields__:
    _CP = dataclasses.replace(_CP, needs_layout_passes=False)
_NW = 32
_EPW = E // _NW
_GB1 = 40


def _sc_gather2(u, v, src, dst):
    @functools.partial(
        pl.kernel,
        out_type=[jax.ShapeDtypeStruct((E, HID), jnp.float32),
                  jax.ShapeDtypeStruct((E, HID), jnp.float32)],
        mesh=_MESH,
        scratch_types=[
            pltpu.VMEM((_GB1,), jnp.int32),
            pltpu.VMEM((_GB1,), jnp.int32),
            pltpu.VMEM((_GB1, HID), jnp.float32),
            pltpu.VMEM((_GB1, HID), jnp.float32),
            pltpu.VMEM((_GB1,), jnp.int32),
            pltpu.VMEM((_GB1,), jnp.int32),
            pltpu.VMEM((_GB1, HID), jnp.float32),
            pltpu.VMEM((_GB1, HID), jnp.float32),
            pltpu.SemaphoreType.DMA,
            pltpu.SemaphoreType.DMA,
            pltpu.SemaphoreType.DMA,
            pltpu.SemaphoreType.DMA,
            pltpu.SemaphoreType.DMA,
            pltpu.SemaphoreType.DMA,
        ],
    )
    def k(u_hbm, v_hbm, src_hbm, dst_hbm, g1_hbm, g2_hbm,
          idxdA, idxsA, urA, vrA, idxdB, idxsB, urB, vrB,
          semIA, semGA, semWA, semIB, semGB, semWB):
        wid = lax.axis_index("s") * 2 + lax.axis_index("c")
        base = wid * _EPW
        nb = _EPW // _GB1

        sets = {
            0: (idxdA, idxsA, urA, vrA, semIA, semGA, semWA),
            1: (idxdB, idxsB, urB, vrB, semIB, semGB, semWB),
        }

        def stage_idx(b, s):
            idxd, idxs, _, _, semI, _, _ = sets[s]
            off = base + b * _GB1
            pltpu.async_copy(dst_hbm.at[pl.ds(off, _GB1)], idxd, semI)
            pltpu.async_copy(src_hbm.at[pl.ds(off, _GB1)], idxs, semI)

        def idx_wait(s):
            idxd, idxs, _, _, semI, _, _ = sets[s]
            pltpu.make_async_copy(dst_hbm.at[pl.ds(0, _GB1)], idxd, semI).wait()
            pltpu.make_async_copy(src_hbm.at[pl.ds(0, _GB1)], idxs, semI).wait()

        def stage_gather(s):
            idxd, idxs, ur, vr, _, semG, _ = sets[s]
            pltpu.async_copy(u_hbm.at[idxd], ur, semG)
            pltpu.async_copy(v_hbm.at[idxs], vr, semG)

        def gather_wait(s):
            idxd, idxs, ur, vr, _, semG, _ = sets[s]
            pltpu.make_async_copy(u_hbm.at[idxd], ur, semG).wait()
            pltpu.make_async_copy(v_hbm.at[idxs], vr, semG).wait()

        def stage_wb(b, s):
            _, _, ur, vr, _, _, semW = sets[s]
            off = base + b * _GB1
            pltpu.async_copy(ur, g1_hbm.at[pl.ds(off, _GB1)], semW)
            pltpu.async_copy(vr, g2_hbm.at[pl.ds(off, _GB1)], semW)

        def wb_wait(s):
            _, _, ur, vr, _, _, semW = sets[s]
            pltpu.make_async_copy(ur, g1_hbm.at[pl.ds(0, _GB1)], semW).wait()
            pltpu.make_async_copy(vr, g2_hbm.at[pl.ds(0, _GB1)], semW).wait()

        stage_idx(0, 0)

        def pair(p, carry):
            b = p * 2

            @pl.when(p > 0)
            def _():
                wb_wait(0)

            idx_wait(0)
            stage_gather(0)
            stage_idx(b + 1, 1)

            @pl.when(p > 0)
            def _():
                wb_wait(1)

            gather_wait(0)
            stage_wb(b, 0)
            idx_wait(1)
            stage_gather(1)
            stage_idx(b + 2, 0)
            gather_wait(1)
            stage_wb(b + 1, 1)
            return carry

        lax.fori_loop(0, nb // 2, pair, jnp.int32(0))
        wb_wait(0)
        idx_wait(0)
        stage_gather(0)
        gather_wait(0)
        stage_wb(nb - 1, 0)
        wb_wait(1)
        wb_wait(0)

    return k(u, v, src, dst)


_NOWN = 320
_NPAD = _NW * _NOWN
_CH = 2000
_GB = 16
_QMAX = 3200
_QTH = _QMAX - _CH - _GB


def _iota16():
    return lax.iota(jnp.int32, 16)


def _lane_splat(v16, k):
    dn = lax.GatherDimensionNumbers(
        offset_dims=(), collapsed_slice_dims=(0,), start_index_map=(0,))
    return lax.gather(v16, jnp.full((16, 1), k, jnp.int32), dn, (1,),
                      mode=lax.GatherScatterMode.PROMISE_IN_BOUNDS)


def _sc_segmax(m, dst):
    @functools.partial(
        pl.kernel,
        out_type=jax.ShapeDtypeStruct((_NPAD, HID), jnp.float32),
        mesh=_MESH,
        compiler_params=_CP,
        scratch_types=[
            pltpu.VMEM((_NOWN + 1, HID), jnp.float32),
            pltpu.VMEM((_CH,), jnp.int32),
            pltpu.VMEM((_CH,), jnp.int32),
            pltpu.VMEM((_QMAX,), jnp.int32),
            pltpu.VMEM((_QMAX,), jnp.int32),
            pltpu.VMEM((_GB, HID), jnp.float32),
            pltpu.VMEM((_GB, HID), jnp.float32),
            pltpu.VMEM((_GB, HID), jnp.float32),
            pltpu.SemaphoreType.DMA,
            pltpu.SemaphoreType.DMA,
            pltpu.SemaphoreType.DMA,
            pltpu.SemaphoreType.DMA,
            pltpu.SemaphoreType.DMA,
        ],
    )
    def k(m_hbm, dst_hbm, out_hbm, acc, dvmA, dvmB, qeid, qlid,
          rows0, rows1, rows2, semA, semB, semG0, semG1, semG2):
        wid = lax.axis_index("s") * 2 + lax.axis_index("c")
        lo = wid * _NOWN
        iot = _iota16()
        ninf = jnp.full((16,), -jnp.inf, jnp.float32)

        @pl.loop(0, _NOWN + 1)
        def _(r):
            rv = jnp.full((16,), 1, jnp.int32) * r

            @pl.loop(0, HID // 16)
            def _(cc):
                plsc.store_scatter(acc, [rv, cc * 16 + iot], ninf)

        def gstart(b, rows, sem):
            pltpu.async_copy(m_hbm.at[qeid.at[pl.ds(b * _GB, _GB)]], rows, sem)

        def gwait(rows, sem):
            pltpu.make_async_copy(m_hbm.at[qeid.at[pl.ds(0, _GB)]], rows, sem).wait()

        def rmw(boff, rows):
            @pl.loop(0, _GB // 2)
            def _(rr):
                r = rr * 2
                one = jnp.full((16,), 1, jnp.int32)
                rsp0 = plsc.load_gather(qlid, [one * (boff + r)])
                rsp1 = plsc.load_gather(qlid, [one * (boff + r + 1)])
                ksp0 = one * r
                ksp1 = one * (r + 1)
                for cc in range(HID // 16):
                    colv = cc * 16 + iot
                    mrow0 = plsc.load_gather(rows, [ksp0, colv])
                    mrow1 = plsc.load_gather(rows, [ksp1, colv])
                    cur0 = plsc.load_gather(acc, [rsp0, colv])
                    plsc.store_scatter(acc, [rsp0, colv], jnp.maximum(cur0, mrow0))
                    cur1 = plsc.load_gather(acc, [rsp1, colv])
                    plsc.store_scatter(acc, [rsp1, colv], jnp.maximum(cur1, mrow1))

        rb = ((rows0, semG0), (rows1, semG1), (rows2, semG2))

        def drain(qn):
            plsc.store_scatter(qeid, [qn + iot], iot * 64)
            plsc.store_scatter(qlid, [qn + iot], jnp.full((16,), _NOWN, jnp.int32))
            nb = (qn + _GB - 1) // _GB
            for j, (rows, sem) in enumerate(rb):
                @pl.when(nb > j)
                def _(rows=rows, sem=sem, j=j):
                    gstart(j, rows, sem)

            def body(p, carry):
                for j, (rows, sem) in enumerate(rb):
                    b = p * 3 + j

                    @pl.when(b < nb)
                    def _(rows=rows, sem=sem, b=b):
                        gwait(rows, sem)
                        rmw(b * _GB, rows)

                        @pl.when(b + 3 < nb)
                        def _():
                            gstart(b + 3, rows, sem)
                return carry

            lax.fori_loop(0, (nb + 2) // 3, body, jnp.int32(0))

        def scanchunk(c, dvm, qn0):
            def scan(i, qn):
                d16 = dvm[pl.ds(i * 16, 16)]
                msk = (d16 >= lo) & (d16 < lo + _NOWN)
                plsc.store_compressed(qeid.at[pl.ds(qn, 16)],
                                      c * _CH + i * 16 + iot, mask=msk)
                plsc.store_compressed(qlid.at[pl.ds(qn, 16)], d16 - lo, mask=msk)
                return qn + plsc.all_reduce_population_count(msk)[0]

            return lax.fori_loop(0, _CH // 16, scan, qn0)

        def maybe_drain(qn):
            @pl.when(qn > _QTH)
            def _():
                drain(qn)

            return jnp.where(qn > _QTH, jnp.int32(0), qn)

        def start(c, dvm, sem):
            pltpu.async_copy(dst_hbm.at[pl.ds(c * _CH, _CH)], dvm, sem)

        def wait(dvm, sem):
            pltpu.make_async_copy(dst_hbm.at[pl.ds(0, _CH)], dvm, sem).wait()

        start(0, dvmA, semA)
        npair = E // _CH // 2

        def pair(p, qn):
            c = p * 2
            start(c + 1, dvmB, semB)
            wait(dvmA, semA)
            qn = scanchunk(c, dvmA, qn)
            qn = maybe_drain(qn)

            @pl.when(p < npair - 1)
            def _():
                start(c + 2, dvmA, semA)

            wait(dvmB, semB)
            qn = scanchunk(c + 1, dvmB, qn)
            return maybe_drain(qn)

        qn = lax.fori_loop(0, npair, pair, jnp.int32(0))

        @pl.when(qn > 0)
        def _():
            drain(qn)

        pltpu.sync_copy(acc.at[pl.ds(0, _NOWN)], out_hbm.at[pl.ds(lo, _NOWN)])

    return k(m, dst)


def _sc_msg(hg_h, als_h, ald_h, src, dst):
    @functools.partial(
        pl.kernel,
        out_type=jax.ShapeDtypeStruct((_NPAD, HID), jnp.float32),
        mesh=_MESH,
        compiler_params=_CP,
        scratch_types=[
            pltpu.VMEM((_NOWN + 1, HID), jnp.float32),
            pltpu.VMEM((N,), jnp.float32),
            pltpu.VMEM((_NOWN,), jnp.float32),
            pltpu.VMEM((16 * _NOWN,), jnp.float32),
            pltpu.VMEM((_NOWN,), jnp.float32),
            pltpu.VMEM((_CH,), jnp.int32),
            pltpu.VMEM((_CH,), jnp.int32),
            pltpu.VMEM((_CH,), jnp.int32),
            pltpu.VMEM((_CH,), jnp.int32),
            pltpu.VMEM((_QMAX,), jnp.int32),
            pltpu.VMEM((_QMAX,), jnp.int32),
            pltpu.VMEM((_QMAX,), jnp.float32),
            pltpu.VMEM((_GB, HID), jnp.float32),
            pltpu.VMEM((_GB, HID), jnp.float32),
            pltpu.VMEM((_GB, HID), jnp.float32),
            pltpu.SemaphoreType.DMA,
            pltpu.SemaphoreType.DMA,
            pltpu.SemaphoreType.DMA,
            pltpu.SemaphoreType.DMA,
            pltpu.SemaphoreType.DMA,
        ],
    )
    def k(hg_hbm, als_hbm, ald_hbm, src_hbm, dst_hbm, out_hbm,
          acc, alsv, aldo, denl, inv, dvmA, svmA, dvmB, svmB,
          qsrc, qlid, qa, rows0, rows1, rows2,
          semA, semB, semG0, semG1, semG2):
        wid = lax.axis_index("s") * 2 + lax.axis_index("c")
        lo = wid * _NOWN
        iot = _iota16()
        zero16 = jnp.zeros((16,), jnp.float32)

        pltpu.sync_copy(als_hbm, alsv)
        pltpu.sync_copy(ald_hbm.at[pl.ds(lo, _NOWN)], aldo)

        @pl.loop(0, _NOWN)
        def _(i):
            denl[pl.ds(i * 16, 16)] = zero16

        @pl.loop(0, _NOWN + 1)
        def _(r):
            rv = jnp.full((16,), 1, jnp.int32) * r

            @pl.loop(0, HID // 16)
            def _(cc):
                plsc.store_scatter(acc, [rv, cc * 16 + iot], zero16)

        def start(c, dvm, svm, sem):
            pltpu.async_copy(dst_hbm.at[pl.ds(c * _CH, _CH)], dvm, sem)
            pltpu.async_copy(src_hbm.at[pl.ds(c * _CH, _CH)], svm, sem)

        def wait(dvm, svm, sem):
            pltpu.make_async_copy(dst_hbm.at[pl.ds(0, _CH)], dvm, sem).wait()
            pltpu.make_async_copy(src_hbm.at[pl.ds(0, _CH)], svm, sem).wait()

        def gstart(b, rows, sem):
            pltpu.async_copy(hg_hbm.at[qsrc.at[pl.ds(b * _GB, _GB)]], rows, sem)

        def gwait(rows, sem):
            pltpu.make_async_copy(hg_hbm.at[qsrc.at[pl.ds(0, _GB)]], rows, sem).wait()

        def rmw(boff, rows):
            @pl.loop(0, _GB // 2)
            def _(rr):
                r = rr * 2
                one = jnp.full((16,), 1, jnp.int32)
                rsp0 = plsc.load_gather(qlid, [one * (boff + r)])
                asp0 = plsc.load_gather(qa, [one * (boff + r)])
                rsp1 = plsc.load_gather(qlid, [one * (boff + r + 1)])
                asp1 = plsc.load_gather(qa, [one * (boff + r + 1)])
                ksp0 = one * r
                ksp1 = one * (r + 1)
                for cc in range(HID // 16):
                    colv = cc * 16 + iot
                    mrow0 = plsc.load_gather(rows, [ksp0, colv])
                    mrow1 = plsc.load_gather(rows, [ksp1, colv])
                    plsc.addupdate_scatter(acc, [rsp0, colv], mrow0 * asp0)
                    plsc.addupdate_scatter(acc, [rsp1, colv], mrow1 * asp1)

        rb = ((rows0, semG0), (rows1, semG1), (rows2, semG2))

        def drain(qn):
            plsc.store_scatter(qsrc, [qn + iot], iot * 64)
            plsc.store_scatter(qlid, [qn + iot], jnp.full((16,), _NOWN, jnp.int32))
            plsc.store_scatter(qa, [qn + iot], zero16)
            nb = (qn + _GB - 1) // _GB
            for j, (rows, sem) in enumerate(rb):
                @pl.when(nb > j)
                def _(rows=rows, sem=sem, j=j):
                    gstart(j, rows, sem)

            def body(p, carry):
                for j, (rows, sem) in enumerate(rb):
                    b = p * 3 + j

                    @pl.when(b < nb)
                    def _(rows=rows, sem=sem, b=b):
                        gwait(rows, sem)
                        rmw(b * _GB, rows)

                        @pl.when(b + 3 < nb)
                        def _():
                            gstart(b + 3, rows, sem)
                return carry

            lax.fori_loop(0, (nb + 2) // 3, body, jnp.int32(0))

        def maybe_drain(qn):
            @pl.when(qn > _QTH)
            def _():
                drain(qn)

            return jnp.where(qn > _QTH, jnp.int32(0), qn)

        def scanchunk(c, dvm, svm, qn0):
            def scan(i, qn):
                d16 = dvm[pl.ds(i * 16, 16)]
                s16 = svm[pl.ds(i * 16, 16)]
                msk = (d16 >= lo) & (d16 < lo + _NOWN)
                lidx = jnp.where(msk, d16 - lo, 0)
                e = plsc.load_gather(alsv, [s16]) + plsc.load_gather(aldo, [lidx])
                e = jnp.where(e >= 0.0, e, 0.2 * e)
                ex = jnp.exp(e)
                plsc.addupdate_scatter(denl, [lidx * 16 + iot], ex, mask=msk)
                plsc.store_compressed(qsrc.at[pl.ds(qn, 16)], s16, mask=msk)
                plsc.store_compressed(qlid.at[pl.ds(qn, 16)], lidx, mask=msk)
                plsc.store_compressed(qa.at[pl.ds(qn, 16)], ex, mask=msk)
                return qn + plsc.all_reduce_population_count(msk)[0]

            return lax.fori_loop(0, _CH // 16, scan, qn0)

        start(0, dvmA, svmA, semA)
        npair = E // _CH // 2

        def pair(p, qn):
            c = p * 2
            start(c + 1, dvmB, svmB, semB)
            wait(dvmA, svmA, semA)
            qn = scanchunk(c, dvmA, svmA, qn)
            qn = maybe_drain(qn)

            @pl.when(p < npair - 1)
            def _():
                start(c + 2, dvmA, svmA, semA)

            wait(dvmB, svmB, semB)
            qn = scanchunk(c + 1, dvmB, svmB, qn)
            return maybe_drain(qn)

        qn = lax.fori_loop(0, npair, pair, jnp.int32(0))

        @pl.when(qn > 0)
        def _():
            drain(qn)

        @pl.loop(0, _NOWN // 16)
        def _(i):
            tot = jnp.full((16,), 1e-16, jnp.float32)
            for l in range(16):
                gidx = (i * 16 + iot) * 16 + l
                tot = tot + plsc.load_gather(denl, [gidx])
            inv[pl.ds(i * 16, 16)] = 1.0 / tot

        @pl.loop(0, _NOWN)
        def _(r):
            rv = jnp.full((16,), 1, jnp.int32) * r
            isp = plsc.load_gather(inv, [rv])
            for cc in range(HID // 16):
                colv = cc * 16 + iot
                mrow = plsc.load_gather(acc, [rv, colv])
                plsc.store_scatter(acc, [rv, colv], mrow * isp)

        pltpu.sync_copy(acc.at[pl.ds(0, _NOWN)], out_hbm.at[pl.ds(lo, _NOWN)])

    return k(hg_h, als_h, ald_h, src, dst)


_NB2 = 2048


def _tc3_body(h_ref, wg_ref, ad_ref, hg0_ref, hg1_ref, al_ref):
    h = h_ref[...]
    h = jnp.where(h > -3e38, h, 0.0)
    hg = jnp.dot(h, wg_ref[...], preferred_element_type=jnp.float32)
    hg0_ref[...] = hg[:, :HID]
    hg1_ref[...] = hg[:, HID:]
    al_ref[...] = jnp.dot(hg, ad_ref[...], preferred_element_type=jnp.float32)


def _tc3(h, wg, ad):
    k = h.shape[1]
    return pl.pallas_call(
        _tc3_body,
        grid=(_NPAD // _NB2,),
        in_specs=[
            pl.BlockSpec((_NB2, k), lambda i: (i, 0)),
            pl.BlockSpec((k, HEADS * HID), lambda i: (0, 0)),
            pl.BlockSpec((HEADS * HID, 2 * HEADS), lambda i: (0, 0)),
        ],
        out_specs=[
            pl.BlockSpec((_NB2, HID), lambda i: (i, 0)),
            pl.BlockSpec((_NB2, HID), lambda i: (i, 0)),
            pl.BlockSpec((_NB2, 2 * HEADS), lambda i: (i, 0)),
        ],
        out_shape=[
            jax.ShapeDtypeStruct((_NPAD, HID), jnp.float32),
            jax.ShapeDtypeStruct((_NPAD, HID), jnp.float32),
            jax.ShapeDtypeStruct((_NPAD, 2 * HEADS), jnp.float32),
        ],
    )(h, wg, ad)


def _tc4_body(a0_ref, a1_ref, bg_ref, wg_ref, ad_ref, h1_ref, hg0_ref, hg1_ref, al_ref):
    agg = jnp.concatenate([a0_ref[...], a1_ref[...]], axis=1)
    h1 = jnp.maximum(agg + bg_ref[...], 0.0)
    h1_ref[...] = h1
    hg = jnp.dot(h1, wg_ref[...], preferred_element_type=jnp.float32)
    hg0_ref[...] = hg[:, :HID]
    hg1_ref[...] = hg[:, HID:]
    al_ref[...] = jnp.dot(hg, ad_ref[...], preferred_element_type=jnp.float32)


def _tc4(a0, a1, bg, wg, ad):
    k = HEADS * HID
    return pl.pallas_call(
        _tc4_body,
        grid=(_NPAD // _NB2,),
        in_specs=[
            pl.BlockSpec((_NB2, HID), lambda i: (i, 0)),
            pl.BlockSpec((_NB2, HID), lambda i: (i, 0)),
            pl.BlockSpec((1, k), lambda i: (0, 0)),
            pl.BlockSpec((k, k), lambda i: (0, 0)),
            pl.BlockSpec((k, 2 * HEADS), lambda i: (0, 0)),
        ],
        out_specs=[
            pl.BlockSpec((_NB2, k), lambda i: (i, 0)),
            pl.BlockSpec((_NB2, HID), lambda i: (i, 0)),
            pl.BlockSpec((_NB2, HID), lambda i: (i, 0)),
            pl.BlockSpec((_NB2, 2 * HEADS), lambda i: (i, 0)),
        ],
        out_shape=[
            jax.ShapeDtypeStruct((_NPAD, k), jnp.float32),
            jax.ShapeDtypeStruct((_NPAD, HID), jnp.float32),
            jax.ShapeDtypeStruct((_NPAD, HID), jnp.float32),
            jax.ShapeDtypeStruct((_NPAD, 2 * HEADS), jnp.float32),
        ],
    )(a0, a1, bg.reshape(1, k), wg, ad)


def _tc5_body(a0_ref, a1_ref, bg_ref, h1_ref, wr_ref, br_ref, out_ref):
    agg = jnp.concatenate([a0_ref[...], a1_ref[...]], axis=1)
    h = jnp.maximum(agg + bg_ref[...], 0.0) + h1_ref[...]
    out_ref[...] = jnp.dot(h, wr_ref[...], preferred_element_type=jnp.float32) + br_ref[...]


def _tc5(a0, a1, bg, h1, wr, br):
    k = HEADS * HID
    return pl.pallas_call(
        _tc5_body,
        grid=(_NPAD // _NB2,),
        in_specs=[
            pl.BlockSpec((_NB2, HID), lambda i: (i, 0)),
            pl.BlockSpec((_NB2, HID), lambda i: (i, 0)),
            pl.BlockSpec((1, k), lambda i: (0, 0)),
            pl.BlockSpec((_NB2, k), lambda i: (i, 0)),
            pl.BlockSpec((k, OUT), lambda i: (0, 0)),
            pl.BlockSpec((1, OUT), lambda i: (0, 0)),
        ],
        out_specs=pl.BlockSpec((_NB2, OUT), lambda i: (i, 0)),
        out_shape=jax.ShapeDtypeStruct((_NPAD, OUT), jnp.float32),
    )(a0, a1, bg.reshape(1, k), h1, wr, br.reshape(1, OUT))


def _attn_mats(a_s, a_d):
    k = HEADS * HID
    ad = jnp.zeros((k, 2 * HEADS), jnp.float32)
    for h in range(HEADS):
        ad = ad.at[h * HID:(h + 1) * HID, h].set(a_s[h])
        ad = ad.at[h * HID:(h + 1) * HID, HEADS + h].set(a_d[h])
    return ad


def kernel(x, edge_index, W1, b1, W2, b2, Wg1, as1, ad1, bg1, Wg2, as2, ad2, bg2, Wr, br):
    src = edge_index[0]
    dst = edge_index[1]

    wu = W1[:F] - W1[F:]
    wv = W1[F:]
    ad1m = _attn_mats(as1, ad1)
    ad2m = _attn_mats(as2, ad2)

    u, v = _tc1(x, wu, wv, b1)
    g1, g2 = _sc_gather2(u, v, src, dst)
    m = _tc2(g1, g2, W2, b2)
    h0p = _sc_segmax(m, dst)

    hg1_0, hg1_1, al1 = _tc3(h0p, Wg1, ad1m)
    a1_0 = _sc_msg(hg1_0, al1[:N, 0], al1[:, 2], src, dst)
    a1_1 = _sc_msg(hg1_1, al1[:N, 1], al1[:, 3], src, dst)

    h1, hg2_0, hg2_1, al2 = _tc4(a1_0, a1_1, bg1, Wg2, ad2m)
    a2_0 = _sc_msg(hg2_0, al2[:N, 0], al2[:, 2], src, dst)
    a2_1 = _sc_msg(hg2_1, al2[:N, 1], al2[:, 3], src, dst)

    return _tc5(a2_0, a2_1, bg2, h1, Wr, br)[:N]
